# Initial kernel scaffold; baseline (speedup 1.0000x reference)
import jax
import jax.numpy as jnp
from jax.experimental import pallas as pl


def kernel(embeddings, current_node):
    raise NotImplementedError("write your pallas kernel here")


# speedup vs baseline: 1.9347x; 1.9347x over previous
import functools

import jax
import jax.numpy as jnp
from jax import lax
from jax.experimental import pallas as pl
from jax.experimental.pallas import tpu as pltpu
from jax.experimental.pallas import tpu_sc as plsc

EMBED_DIM = 128
BATCH = 1024
NUM_LOC = 1000

_INFO = plsc.get_sparse_core_info()
_NC = _INFO.num_cores
_NS = _INFO.num_subcores
_L = _INFO.num_lanes
_NW = _NC * _NS
_B_PER_W = BATCH // _NW

_MESH = plsc.VectorSubcoreMesh(core_axis_name="c", subcore_axis_name="s")


@functools.partial(
    pl.kernel,
    mesh=_MESH,
    out_type=jax.ShapeDtypeStruct((BATCH, EMBED_DIM), jnp.float32),
    scratch_types=[
        pltpu.VMEM((_B_PER_W,), jnp.int32),
        pltpu.VMEM((_B_PER_W, EMBED_DIM), jnp.float32),
        pltpu.SemaphoreType.DMA,
    ],
)
def _gather_rows(table_hbm, idx_hbm, out_hbm, idx_v, rows_v, sem):
    wid = lax.axis_index("s") * _NC + lax.axis_index("c")
    base = wid * _B_PER_W
    pltpu.sync_copy(idx_hbm.at[pl.ds(base, _B_PER_W)], idx_v)
    lane = lax.iota(jnp.int32, _L)
    for j in range(0, _B_PER_W, _L):
        row_base = (base + j + lane) * NUM_LOC
        idx_v[pl.ds(j, _L)] = idx_v[pl.ds(j, _L)] + row_base
    pltpu.async_copy(table_hbm.at[idx_v], rows_v, sem).wait()
    pltpu.sync_copy(rows_v, out_hbm.at[pl.ds(base, _B_PER_W)])


def kernel(embeddings, current_node):
    table = embeddings.reshape(BATCH * NUM_LOC, EMBED_DIM)
    idx = current_node.astype(jnp.int32)
    out = _gather_rows(table, idx)
    return out[:, None, :]


# speedup vs baseline: 1.9501x; 1.0080x over previous
import functools

import jax
import jax.numpy as jnp
from jax import lax
from jax.experimental import pallas as pl
from jax.experimental.pallas import tpu as pltpu
from jax.experimental.pallas import tpu_sc as plsc

EMBED_DIM = 128
BATCH = 1024
NUM_LOC = 1000

_INFO = plsc.get_sparse_core_info()
_NC = _INFO.num_cores
_NS = _INFO.num_subcores
_L = _INFO.num_lanes
_NW = _NC * _NS
_B_PER_W = BATCH // _NW

_MESH = plsc.VectorSubcoreMesh(core_axis_name="c", subcore_axis_name="s")


@functools.partial(
    pl.kernel,
    mesh=_MESH,
    out_type=jax.ShapeDtypeStruct((BATCH, EMBED_DIM), jnp.float32),
    scratch_types=[
        pltpu.VMEM((_L,), jnp.int32),
        pltpu.VMEM((_L,), jnp.int32),
        pltpu.VMEM((_B_PER_W,), jnp.int32),
        pltpu.VMEM((_L, EMBED_DIM), jnp.float32),
        pltpu.VMEM((_L, EMBED_DIM), jnp.float32),
        pltpu.SemaphoreType.DMA,
        pltpu.SemaphoreType.DMA,
        pltpu.SemaphoreType.DMA,
    ],
)
def _gather_rows(table_hbm, idx_hbm, out_hbm,
                 idx_a, idx_b, idx_raw, rows_a, rows_b, sem_a, sem_b, sem_w):
    wid = lax.axis_index("s") * _NC + lax.axis_index("c")
    base = wid * _B_PER_W
    pltpu.sync_copy(idx_hbm.at[pl.ds(base, _B_PER_W)], idx_raw)
    lane = lax.iota(jnp.int32, _L)
    idx_a[...] = idx_raw[pl.ds(0, _L)] + (base + lane) * NUM_LOC
    idx_b[...] = idx_raw[pl.ds(_L, _L)] + (base + _L + lane) * NUM_LOC
    ga = pltpu.async_copy(table_hbm.at[idx_a], rows_a, sem_a)
    gb = pltpu.async_copy(table_hbm.at[idx_b], rows_b, sem_b)
    ga.wait()
    wa = pltpu.async_copy(rows_a, out_hbm.at[pl.ds(base, _L)], sem_w)
    gb.wait()
    wb = pltpu.async_copy(rows_b, out_hbm.at[pl.ds(base + _L, _L)], sem_w)
    wa.wait()
    wb.wait()


def kernel(embeddings, current_node):
    table = embeddings.reshape(BATCH * NUM_LOC, EMBED_DIM)
    idx = current_node.astype(jnp.int32)
    out = _gather_rows(table, idx)
    return out[:, None, :]
